# baseline (speedup 1.0000x reference)
import jax
import jax.numpy as jnp
from jax.experimental import pallas as pl
from jax.experimental.pallas import tpu as pltpu

_TOP_K = 8
_BT = 512

_ERFC_P = [+2.326819970068386e-2, -1.387039388740657e-1,
           +3.687424674597105e-1, -5.824733027278666e-1,
           +6.210004621745983e-1, -4.944515323274145e-1,
           +3.404879937665872e-1, -2.741127028184656e-1,
           +5.638259427386472e-1]
_ERFC_R = [-1.047766399936249e+1, +1.297719955372516e+1,
           -7.495518717768503e+0, +2.921019019210786e+0,
           -1.015265279202700e+0, +4.218463358204948e-1,
           -2.820767439740514e-1, +5.641895067754075e-1]
_ERF_T = [+7.853861353153693e-5, -8.010193625184903e-4,
          +5.188327685732524e-3, -2.685381193529856e-2,
          +1.128358514861418e-1, -3.761262582423300e-1,
          +1.128379165726710e+0]


def _poly(cs, y):
    acc = jnp.full_like(y, jnp.float32(cs[0]))
    for c in cs[1:]:
        acc = acc * y + jnp.float32(c)
    return acc


def _erfc(x):
    x2 = x * x
    z = jnp.exp(-x2)
    ax = jnp.abs(x)
    q = 1.0 / ax
    y = 1.0 / x2
    p = jnp.where(ax < 2.0, _poly(_ERFC_P, y), _poly(_ERFC_R, y))
    ya = (z * q) * p
    ya = jnp.where(z == 0.0, 0.0, ya)
    big = jnp.where(x < 0.0, 2.0 - ya, ya)
    small = 1.0 - x * _poly(_ERF_T, x2)
    return jnp.where(ax < 1.0, small, big)


def _gelu(h):
    return 0.5 * h * _erfc(-h * 0.7071067811865476)


def _layer_norm_t(h, g_col, be_col):
    n = h.shape[0]
    mu = jnp.sum(h, axis=0, keepdims=True) / n
    c = h - mu
    var = jnp.sum(c * c, axis=0, keepdims=True) / n
    return c / jnp.sqrt(var + 1e-5) * g_col + be_col


_CHUNKS = 2


def _router_kernel(x_ref, w1_ref, b1_ref, g1_ref, be1_ref,
                   w2_ref, b2_ref, g2_ref, be2_ref,
                   w3_ref, b3_ref,
                   w_out_ref, idx_out_ref, topw_out_ref):
    bt = x_ref.shape[0]
    half = bt // _CHUNKS
    for part in range(_CHUNKS):
        sl = slice(part * half, (part + 1) * half)
        x = x_ref[sl, :]

        h = jax.lax.dot_general(w1_ref[...], x, (((1,), (1,)), ((), ())),
                                preferred_element_type=jnp.float32)
        h = h + b1_ref[...]
        h = _gelu(_layer_norm_t(h, g1_ref[...], be1_ref[...]))

        h = jax.lax.dot_general(w2_ref[...], h, (((1,), (0,)), ((), ())),
                                preferred_element_type=jnp.float32)
        h = h + b2_ref[...]
        h = _gelu(_layer_norm_t(h, g2_ref[...], be2_ref[...]))

        logits = jax.lax.dot_general(w3_ref[...], h, (((1,), (0,)), ((), ())),
                                     preferred_element_type=jnp.float32)
        logits = logits + b3_ref[...]
        logits = logits - jnp.max(logits, axis=0, keepdims=True)
        ex = jnp.exp(logits)
        w = ex / jnp.sum(ex, axis=0, keepdims=True)
        w_out_ref[:, sl] = w

        iota = jax.lax.broadcasted_iota(jnp.int32, w.shape, 0)
        cur = w
        idx_rows = []
        val_rows = []
        for _ in range(_TOP_K):
            m = jnp.max(cur, axis=0, keepdims=True)
            is_max = cur == m
            idx = jnp.min(jnp.where(is_max, iota, w.shape[0]), axis=0,
                          keepdims=True)
            val_rows.append(m)
            idx_rows.append(idx)
            cur = jnp.where(iota == idx, -1.0, cur)
        topw = jnp.concatenate(val_rows, axis=0)
        topi = jnp.concatenate(idx_rows, axis=0)
        topw = topw / jnp.sum(topw, axis=0, keepdims=True)
        idx_out_ref[:, sl] = topi
        topw_out_ref[:, sl] = topw


def kernel(features, W1, b1, g1, be1, W2, b2, g2, be2, W3, b3):
    B, D = features.shape
    H1 = W1.shape[0]
    H2 = W2.shape[0]
    E = W3.shape[0]
    bt = min(_BT, B)
    grid = (B // bt,)

    def tok(i):
        return (i, 0)

    def tok_col(i):
        return (0, i)

    def fixed(i):
        return (0, 0)

    col = lambda v: v.reshape(-1, 1)

    out = pl.pallas_call(
        _router_kernel,
        grid=grid,
        in_specs=[
            pl.BlockSpec((bt, D), tok),
            pl.BlockSpec((H1, D), fixed),
            pl.BlockSpec((H1, 1), fixed),
            pl.BlockSpec((H1, 1), fixed),
            pl.BlockSpec((H1, 1), fixed),
            pl.BlockSpec((H2, H1), fixed),
            pl.BlockSpec((H2, 1), fixed),
            pl.BlockSpec((H2, 1), fixed),
            pl.BlockSpec((H2, 1), fixed),
            pl.BlockSpec((E, H2), fixed),
            pl.BlockSpec((E, 1), fixed),
        ],
        out_specs=[
            pl.BlockSpec((E, bt), tok_col),
            pl.BlockSpec((_TOP_K, bt), tok_col),
            pl.BlockSpec((_TOP_K, bt), tok_col),
        ],
        out_shape=[
            jax.ShapeDtypeStruct((E, B), jnp.float32),
            jax.ShapeDtypeStruct((_TOP_K, B), jnp.int32),
            jax.ShapeDtypeStruct((_TOP_K, B), jnp.float32),
        ],
        compiler_params=pltpu.CompilerParams(
            dimension_semantics=("parallel",),
        ),
    )(features, W1, col(b1), col(g1), col(be1),
      W2, col(b2), col(g2), col(be2), W3, col(b3))
    return (out[0].T, out[1].T, out[2].T)


# speedup vs baseline: 1.3127x; 1.3127x over previous
import jax
import jax.numpy as jnp
from jax.experimental import pallas as pl
from jax.experimental.pallas import tpu as pltpu

_TOP_K = 8
_BT = 1024

_ERFC_P = [+2.326819970068386e-2, -1.387039388740657e-1,
           +3.687424674597105e-1, -5.824733027278666e-1,
           +6.210004621745983e-1, -4.944515323274145e-1,
           +3.404879937665872e-1, -2.741127028184656e-1,
           +5.638259427386472e-1]
_ERFC_R = [-1.047766399936249e+1, +1.297719955372516e+1,
           -7.495518717768503e+0, +2.921019019210786e+0,
           -1.015265279202700e+0, +4.218463358204948e-1,
           -2.820767439740514e-1, +5.641895067754075e-1]
_ERF_T = [+7.853861353153693e-5, -8.010193625184903e-4,
          +5.188327685732524e-3, -2.685381193529856e-2,
          +1.128358514861418e-1, -3.761262582423300e-1,
          +1.128379165726710e+0]


def _poly(cs, y):
    acc = jnp.full_like(y, jnp.float32(cs[0]))
    for c in cs[1:]:
        acc = acc * y + jnp.float32(c)
    return acc


def _erfc(x):
    x2 = x * x
    z = jnp.exp(-x2)
    ax = jnp.abs(x)
    q = 1.0 / ax
    y = 1.0 / x2
    p = jnp.where(ax < 2.0, _poly(_ERFC_P, y), _poly(_ERFC_R, y))
    ya = (z * q) * p
    ya = jnp.where(z == 0.0, 0.0, ya)
    big = jnp.where(x < 0.0, 2.0 - ya, ya)
    small = 1.0 - x * _poly(_ERF_T, x2)
    return jnp.where(ax < 1.0, small, big)


def _gelu(h):
    return 0.5 * h * _erfc(-h * 0.7071067811865476)


def _layer_norm_t(h, g_col, be_col):
    n = h.shape[0]
    mu = jnp.sum(h, axis=0, keepdims=True) / n
    c = h - mu
    var = jnp.sum(c * c, axis=0, keepdims=True) / n
    return c / jnp.sqrt(var + 1e-5) * g_col + be_col


_CHUNKS = 1


def _router_kernel(x_ref, w1_ref, b1_ref, g1_ref, be1_ref,
                   w2_ref, b2_ref, g2_ref, be2_ref,
                   w3_ref, b3_ref,
                   w_out_ref, idx_out_ref, topw_out_ref):
    bt = x_ref.shape[0]
    half = bt // _CHUNKS
    for part in range(_CHUNKS):
        sl = slice(part * half, (part + 1) * half)
        x = x_ref[sl, :]

        h = jax.lax.dot_general(w1_ref[...], x, (((1,), (1,)), ((), ())),
                                preferred_element_type=jnp.float32)
        h = h + b1_ref[...]
        h = _gelu(_layer_norm_t(h, g1_ref[...], be1_ref[...]))

        h = jax.lax.dot_general(w2_ref[...], h, (((1,), (0,)), ((), ())),
                                preferred_element_type=jnp.float32)
        h = h + b2_ref[...]
        h = _gelu(_layer_norm_t(h, g2_ref[...], be2_ref[...]))

        logits = jax.lax.dot_general(w3_ref[...], h, (((1,), (0,)), ((), ())),
                                     preferred_element_type=jnp.float32)
        logits = logits + b3_ref[...]
        logits = logits - jnp.max(logits, axis=0, keepdims=True)
        ex = jnp.exp(logits)
        w = ex / jnp.sum(ex, axis=0, keepdims=True)
        w_out_ref[:, sl] = w

        iota = jax.lax.broadcasted_iota(jnp.int32, w.shape, 0)
        cur = w
        idx_rows = []
        val_rows = []
        for _ in range(_TOP_K):
            m = jnp.max(cur, axis=0, keepdims=True)
            is_max = cur == m
            idx = jnp.min(jnp.where(is_max, iota, w.shape[0]), axis=0,
                          keepdims=True)
            val_rows.append(m)
            idx_rows.append(idx)
            cur = jnp.where(iota == idx, -1.0, cur)
        topw = jnp.concatenate(val_rows, axis=0)
        topi = jnp.concatenate(idx_rows, axis=0)
        topw = topw / jnp.sum(topw, axis=0, keepdims=True)
        idx_out_ref[:, sl] = topi
        topw_out_ref[:, sl] = topw


def kernel(features, W1, b1, g1, be1, W2, b2, g2, be2, W3, b3):
    B, D = features.shape
    H1 = W1.shape[0]
    H2 = W2.shape[0]
    E = W3.shape[0]
    bt = min(_BT, B)
    grid = (B // bt,)

    def tok(i):
        return (i, 0)

    def tok_col(i):
        return (0, i)

    def fixed(i):
        return (0, 0)

    col = lambda v: v.reshape(-1, 1)

    out = pl.pallas_call(
        _router_kernel,
        grid=grid,
        in_specs=[
            pl.BlockSpec((bt, D), tok),
            pl.BlockSpec((H1, D), fixed),
            pl.BlockSpec((H1, 1), fixed),
            pl.BlockSpec((H1, 1), fixed),
            pl.BlockSpec((H1, 1), fixed),
            pl.BlockSpec((H2, H1), fixed),
            pl.BlockSpec((H2, 1), fixed),
            pl.BlockSpec((H2, 1), fixed),
            pl.BlockSpec((H2, 1), fixed),
            pl.BlockSpec((E, H2), fixed),
            pl.BlockSpec((E, 1), fixed),
        ],
        out_specs=[
            pl.BlockSpec((E, bt), tok_col),
            pl.BlockSpec((_TOP_K, bt), tok_col),
            pl.BlockSpec((_TOP_K, bt), tok_col),
        ],
        out_shape=[
            jax.ShapeDtypeStruct((E, B), jnp.float32),
            jax.ShapeDtypeStruct((_TOP_K, B), jnp.int32),
            jax.ShapeDtypeStruct((_TOP_K, B), jnp.float32),
        ],
        compiler_params=pltpu.CompilerParams(
            dimension_semantics=("parallel",),
        ),
    )(features, W1, col(b1), col(g1), col(be1),
      W2, col(b2), col(g2), col(be2), W3, col(b3))
    return (out[0].T, out[1].T, out[2].T)


# speedup vs baseline: 1.3770x; 1.0489x over previous
import jax
import jax.numpy as jnp
from jax.experimental import pallas as pl
from jax.experimental.pallas import tpu as pltpu

_TOP_K = 8
_BT = 2048

_ERFC_P = [+2.326819970068386e-2, -1.387039388740657e-1,
           +3.687424674597105e-1, -5.824733027278666e-1,
           +6.210004621745983e-1, -4.944515323274145e-1,
           +3.404879937665872e-1, -2.741127028184656e-1,
           +5.638259427386472e-1]
_ERFC_R = [-1.047766399936249e+1, +1.297719955372516e+1,
           -7.495518717768503e+0, +2.921019019210786e+0,
           -1.015265279202700e+0, +4.218463358204948e-1,
           -2.820767439740514e-1, +5.641895067754075e-1]
_ERF_T = [+7.853861353153693e-5, -8.010193625184903e-4,
          +5.188327685732524e-3, -2.685381193529856e-2,
          +1.128358514861418e-1, -3.761262582423300e-1,
          +1.128379165726710e+0]


def _poly(cs, y):
    acc = jnp.full_like(y, jnp.float32(cs[0]))
    for c in cs[1:]:
        acc = acc * y + jnp.float32(c)
    return acc


def _erfc(x):
    x2 = x * x
    z = jnp.exp(-x2)
    ax = jnp.abs(x)
    q = 1.0 / ax
    y = 1.0 / x2
    p = jnp.where(ax < 2.0, _poly(_ERFC_P, y), _poly(_ERFC_R, y))
    ya = (z * q) * p
    ya = jnp.where(z == 0.0, 0.0, ya)
    big = jnp.where(x < 0.0, 2.0 - ya, ya)
    small = 1.0 - x * _poly(_ERF_T, x2)
    return jnp.where(ax < 1.0, small, big)


def _gelu(h):
    return 0.5 * h * _erfc(-h * 0.7071067811865476)


def _layer_norm_t(h, g_col, be_col):
    n = h.shape[0]
    mu = jnp.sum(h, axis=0, keepdims=True) / n
    c = h - mu
    var = jnp.sum(c * c, axis=0, keepdims=True) / n
    return c / jnp.sqrt(var + 1e-5) * g_col + be_col


_CHUNKS = 1


def _router_kernel(x_ref, w1_ref, b1_ref, g1_ref, be1_ref,
                   w2_ref, b2_ref, g2_ref, be2_ref,
                   w3_ref, b3_ref,
                   w_out_ref, idx_out_ref, topw_out_ref):
    bt = x_ref.shape[0]
    half = bt // _CHUNKS
    for part in range(_CHUNKS):
        sl = slice(part * half, (part + 1) * half)
        x = x_ref[sl, :]

        h = jax.lax.dot_general(w1_ref[...], x, (((1,), (1,)), ((), ())),
                                preferred_element_type=jnp.float32)
        h = h + b1_ref[...]
        h = _gelu(_layer_norm_t(h, g1_ref[...], be1_ref[...]))

        h = jax.lax.dot_general(w2_ref[...], h, (((1,), (0,)), ((), ())),
                                preferred_element_type=jnp.float32)
        h = h + b2_ref[...]
        h = _gelu(_layer_norm_t(h, g2_ref[...], be2_ref[...]))

        logits = jax.lax.dot_general(w3_ref[...], h, (((1,), (0,)), ((), ())),
                                     preferred_element_type=jnp.float32)
        logits = logits + b3_ref[...]
        logits = logits - jnp.max(logits, axis=0, keepdims=True)
        ex = jnp.exp(logits)
        w = ex / jnp.sum(ex, axis=0, keepdims=True)
        w_out_ref[:, sl] = w

        iota = jax.lax.broadcasted_iota(jnp.int32, w.shape, 0)
        cur = w
        idx_rows = []
        val_rows = []
        for _ in range(_TOP_K):
            m = jnp.max(cur, axis=0, keepdims=True)
            is_max = cur == m
            idx = jnp.min(jnp.where(is_max, iota, w.shape[0]), axis=0,
                          keepdims=True)
            val_rows.append(m)
            idx_rows.append(idx)
            cur = jnp.where(iota == idx, -1.0, cur)
        topw = jnp.concatenate(val_rows, axis=0)
        topi = jnp.concatenate(idx_rows, axis=0)
        topw = topw / jnp.sum(topw, axis=0, keepdims=True)
        idx_out_ref[:, sl] = topi
        topw_out_ref[:, sl] = topw


def kernel(features, W1, b1, g1, be1, W2, b2, g2, be2, W3, b3):
    B, D = features.shape
    H1 = W1.shape[0]
    H2 = W2.shape[0]
    E = W3.shape[0]
    bt = min(_BT, B)
    grid = (B // bt,)

    def tok(i):
        return (i, 0)

    def tok_col(i):
        return (0, i)

    def fixed(i):
        return (0, 0)

    col = lambda v: v.reshape(-1, 1)

    out = pl.pallas_call(
        _router_kernel,
        grid=grid,
        in_specs=[
            pl.BlockSpec((bt, D), tok),
            pl.BlockSpec((H1, D), fixed),
            pl.BlockSpec((H1, 1), fixed),
            pl.BlockSpec((H1, 1), fixed),
            pl.BlockSpec((H1, 1), fixed),
            pl.BlockSpec((H2, H1), fixed),
            pl.BlockSpec((H2, 1), fixed),
            pl.BlockSpec((H2, 1), fixed),
            pl.BlockSpec((H2, 1), fixed),
            pl.BlockSpec((E, H2), fixed),
            pl.BlockSpec((E, 1), fixed),
        ],
        out_specs=[
            pl.BlockSpec((E, bt), tok_col),
            pl.BlockSpec((_TOP_K, bt), tok_col),
            pl.BlockSpec((_TOP_K, bt), tok_col),
        ],
        out_shape=[
            jax.ShapeDtypeStruct((E, B), jnp.float32),
            jax.ShapeDtypeStruct((_TOP_K, B), jnp.int32),
            jax.ShapeDtypeStruct((_TOP_K, B), jnp.float32),
        ],
        compiler_params=pltpu.CompilerParams(
            dimension_semantics=("parallel",),
        ),
    )(features, W1, col(b1), col(g1), col(be1),
      W2, col(b2), col(g2), col(be2), W3, col(b3))
    return (out[0].T, out[1].T, out[2].T)
